# baseline (speedup 1.0000x reference)
import jax
import jax.numpy as jnp
from jax.experimental import pallas as pl

N = 10000
B = 16
IN_DIM = 2
HID = 64
K_DIFF = 2


def _spmm(row, col, val, m):
    return jax.ops.segment_sum(val[:, None] * m[col], row, num_segments=N)


def _matmul_kernel(x_ref, w_ref, b_ref, o_ref):
    o_ref[...] = jnp.tanh(
        jnp.dot(x_ref[...], w_ref[...], preferred_element_type=jnp.float32)
        + b_ref[...]
    )


def kernel(inputs, state_t, s1_row, s1_col, s1_val, s2_row, s2_col, s2_val, weights, biases):
    Bb, Nn, _ = inputs.shape
    x_cat = jnp.concatenate([inputs, state_t], axis=2)
    in_size = x_cat.shape[2]
    x0 = jnp.transpose(x_cat, (1, 2, 0)).reshape(Nn, -1)
    xs = [x0]
    for row, col, val in ((s1_row, s1_col, s1_val), (s2_row, s2_col, s2_val)):
        x1 = _spmm(row, col, val, x0)
        xs.append(x1)
        xk_2, xk_1 = x0, x1
        for _ in range(2, K_DIFF + 1):
            x2 = 2.0 * _spmm(row, col, val, xk_1) - xk_2
            xs.append(x2)
            xk_2, xk_1 = xk_1, x2
    x = jnp.stack(xs, axis=0)
    num_m = x.shape[0]
    x = x.reshape(num_m, Nn, in_size, Bb)
    x = jnp.transpose(x, (3, 1, 2, 0))
    x = x.reshape(Bb * Nn, in_size * num_m)

    M = Bb * Nn
    BLK = 640
    out = pl.pallas_call(
        _matmul_kernel,
        grid=(M // BLK,),
        in_specs=[
            pl.BlockSpec((BLK, in_size * num_m), lambda i: (i, 0)),
            pl.BlockSpec((in_size * num_m, HID), lambda i: (0, 0)),
            pl.BlockSpec((HID,), lambda i: (0,)),
        ],
        out_specs=pl.BlockSpec((BLK, HID), lambda i: (i, 0)),
        out_shape=jax.ShapeDtypeStruct((M, HID), jnp.float32),
    )(x, weights, biases)
    return out.reshape(Bb, Nn, HID)


# speedup vs baseline: 1.8693x; 1.8693x over previous
import jax
import jax.numpy as jnp
from jax import lax
from jax.experimental import pallas as pl
from jax.experimental.pallas import tpu as pltpu
from jax.experimental.pallas import tpu_sc as plsc

N = 10000
NP = 10240
B = 16
HID = 64
PADW = 80
E = 160000
NC = 2
NS = 16
EPT = E // NS
G = 80
NBLK = EPT // G
RPT = NP // NS
BPC = B // NC
ZR = 80
UNROLL = 10
NVR = PADW // 16


def _sc_body(x0_ref, c1_ref, r1_ref, v1_ref, c2_ref, r2_ref, v2_ref,
             y1_ref, y2_ref, y3_ref, y4_ref,
             col_v, row_v, val_v, idx_v, stage, zbuf, acc, sem):
    c = lax.axis_index("c")
    s = lax.axis_index("s")

    zero16 = jnp.zeros((16,), jnp.float32)

    def zb(i, _):
        for r in range(NVR):
            zbuf[i, pl.ds(r * 16, 16)] = zero16
        return 0
    lax.fori_loop(0, ZR, zb, 0)

    def spmm_pass(src_ref, dst_ref, b):
        for z in range(RPT // ZR):
            pltpu.sync_copy(zbuf, acc.at[pl.ds(s * RPT + z * ZR, ZR)])
        plsc.subcore_barrier()

        def blk(k, _):
            pltpu.async_copy(
                src_ref.at[idx_v.at[pl.ds(k * G, G)]], stage, sem).wait()

            def grp(g, _):
                chunk = val_v[pl.ds(k * G + g * 16, 16)]
                for u in range(16):
                    vv = jnp.broadcast_to(chunk[u], (16,))
                    j = g * 16 + u
                    for r in range(NVR):
                        stage[j, pl.ds(r * 16, 16)] = (
                            stage[j, pl.ds(r * 16, 16)] * vv)
                return 0
            lax.fori_loop(0, G // 16, grp, 0)
            pltpu.sync_copy(stage, acc.at[row_v.at[k]], add=True)
            return 0
        lax.fori_loop(0, NBLK, blk, 0)
        plsc.subcore_barrier()
        pltpu.sync_copy(acc.at[pl.ds(s * RPT, RPT)],
                        dst_ref.at[pl.ds(b * NP + s * RPT, RPT)])

    for (ch, rh, vh, dst_a, dst_b) in (
            (c1_ref, r1_ref, v1_ref, y1_ref, y2_ref),
            (c2_ref, r2_ref, v2_ref, y3_ref, y4_ref)):
        pltpu.sync_copy(ch.at[s], col_v)
        pltpu.sync_copy(rh.at[s], row_v)
        pltpu.sync_copy(vh.at[s], val_v)

        def batch_body(bi, _):
            b = c * BPC + bi
            off = b * NP

            def ix(i, _):
                sl = pl.ds(i * 16, 16)
                idx_v[sl] = col_v[sl] + off
                return 0
            lax.fori_loop(0, EPT // 16, ix, 0)
            spmm_pass(x0_ref, dst_a, b)
            spmm_pass(dst_a, dst_b, b)
            return 0
        lax.fori_loop(0, BPC, batch_body, 0)


def _mm_body(x0_ref, y1_ref, y2_ref, y3_ref, y4_ref, w_ref, b_ref, o_ref):
    acc = jnp.dot(x0_ref[0], w_ref[0], preferred_element_type=jnp.float32)
    acc += jnp.dot(y1_ref[0], w_ref[1], preferred_element_type=jnp.float32)
    acc += jnp.dot(y2_ref[0], w_ref[2], preferred_element_type=jnp.float32)
    acc += jnp.dot(y3_ref[0], w_ref[3], preferred_element_type=jnp.float32)
    acc += jnp.dot(y4_ref[0], w_ref[4], preferred_element_type=jnp.float32)
    o_ref[0] = jnp.tanh(acc + b_ref[...])


def kernel(inputs, state_t, s1_row, s1_col, s1_val, s2_row, s2_col, s2_val,
           weights, biases):
    Bb, Nn, in_dim = inputs.shape
    x_cat = jnp.concatenate([inputs, state_t], axis=2)
    in_size = x_cat.shape[2]
    x0p = jnp.pad(x_cat, ((0, 0), (0, NP - Nn), (0, PADW - in_size)))
    x0f = x0p.reshape(Bb * NP, PADW)

    c1 = s1_col.reshape(NS, EPT)
    r1 = s1_row.reshape(NS, NBLK, G)
    v1 = s1_val.reshape(NS, EPT)
    c2 = s2_col.reshape(NS, EPT)
    r2 = s2_row.reshape(NS, NBLK, G)
    v2 = s2_val.reshape(NS, EPT)

    mesh = plsc.VectorSubcoreMesh(core_axis_name="c", subcore_axis_name="s")
    sc = pl.kernel(
        _sc_body,
        out_type=[jax.ShapeDtypeStruct((Bb * NP, PADW), jnp.float32)] * 4,
        mesh=mesh,
        compiler_params=pltpu.CompilerParams(use_tc_tiling_on_sc=False),
        scratch_types=[
            pltpu.VMEM((EPT,), jnp.int32),
            pltpu.VMEM((NBLK, G), jnp.int32),
            pltpu.VMEM((EPT,), jnp.float32),
            pltpu.VMEM((EPT,), jnp.int32),
            pltpu.VMEM((G, PADW), jnp.float32),
            pltpu.VMEM((ZR, PADW), jnp.float32),
            pltpu.VMEM_SHARED((NP, PADW), jnp.float32),
            pltpu.SemaphoreType.DMA,
        ],
    )
    y1, y2, y3, y4 = sc(x0f, c1, r1, v1, c2, r2, v2)

    wm = weights.reshape(in_size, 5, HID)
    wa = jnp.stack([wm[:, 0] - wm[:, 2] - wm[:, 4], wm[:, 1], 2.0 * wm[:, 2],
                    wm[:, 3], 2.0 * wm[:, 4]], axis=0)
    wp = jnp.pad(wa, ((0, 0), (0, PADW - in_size), (0, 0)))

    NB = 1000
    feat_spec = pl.BlockSpec((1, NB, PADW), lambda bb, nn: (bb, nn, 0))
    out = pl.pallas_call(
        _mm_body,
        grid=(Bb, Nn // NB),
        in_specs=[feat_spec] * 5 + [
            pl.BlockSpec((5, PADW, HID), lambda bb, nn: (0, 0, 0)),
            pl.BlockSpec((HID,), lambda bb, nn: (0,)),
        ],
        out_specs=pl.BlockSpec((1, NB, HID), lambda bb, nn: (bb, nn, 0)),
        out_shape=jax.ShapeDtypeStruct((Bb, Nn, HID), jnp.float32),
    )(x0p, y1.reshape(Bb, NP, PADW), y2.reshape(Bb, NP, PADW),
      y3.reshape(Bb, NP, PADW), y4.reshape(Bb, NP, PADW), wp, biases)
    return out


# speedup vs baseline: 2.0673x; 1.1059x over previous
import jax
import jax.numpy as jnp
from jax import lax
from jax.experimental import pallas as pl
from jax.experimental.pallas import tpu as pltpu
from jax.experimental.pallas import tpu_sc as plsc

N = 10000
NP = 10240
B = 16
HID = 64
PADW = 80
E = 160000
NC = 2
NS = 16
EPT = E // NS
EPTP = 10240
G = 128
NBLK = EPTP // G
NSTG = 4
RPT = NP // NS
BPC = B // NC
NVR = PADW // 16


def _sc_body(x0_ref, c1_ref, r1_ref, v1_ref, c2_ref, r2_ref, v2_ref, z_ref,
             y1_ref, y2_ref, y3_ref, y4_ref,
             col_v, row_v, val_v, idx0, idx1, idx2, idx3,
             st0, st1, st2, st3, acc,
             gsem0, gsem1, gsem2, gsem3, ssem0, ssem1, ssem2, ssem3):
    c = lax.axis_index("c")
    s = lax.axis_index("s")
    stages = (st0, st1, st2, st3)
    idxs = (idx0, idx1, idx2, idx3)
    gsems = (gsem0, gsem1, gsem2, gsem3)
    ssems = (ssem0, ssem1, ssem2, ssem3)

    def mk_idx(p, k, off):
        def ix(i, _):
            sl = pl.ds(i * 16, 16)
            idxs[p][sl] = col_v[pl.ds(k * G + i * 16, 16)] + off
            return 0
        lax.fori_loop(0, G // 16, ix, 0)

    def scale(p):
        st = stages[p]

        def grp(g, kG):
            chunk = val_v[pl.ds(kG + g * 16, 16)]
            for u in range(16):
                vv = jnp.broadcast_to(chunk[u], (16,))
                j = g * 16 + u
                for r in range(NVR):
                    st[j, pl.ds(r * 16, 16)] = st[j, pl.ds(r * 16, 16)] * vv
            return kG
        return grp

    def spmm_pass(src_ref, dst_ref, b):
        pltpu.sync_copy(z_ref, acc.at[pl.ds(s * RPT, RPT)])
        plsc.subcore_barrier()

        off = b * NP
        mk_idx(0, 0, off)
        pltpu.async_copy(src_ref.at[idx0], st0, gsem0)

        def blk(m, _):
            for u in range(NSTG):
                k = m * NSTG + u
                q = (u + 1) % NSTG

                @pl.when(k + 1 < NBLK)
                def _prefetch():
                    mk_idx(q, k + 1, off)

                    @pl.when(k >= NSTG - 1)
                    def _drain_prev_scatter():
                        pltpu.make_async_copy(
                            stages[q], acc.at[row_v.at[k - (NSTG - 1)]],
                            ssems[q]).wait()
                    pltpu.async_copy(src_ref.at[idxs[q]], stages[q],
                                     gsems[q])

                pltpu.make_async_copy(src_ref.at[idxs[u]], stages[u],
                                      gsems[u]).wait()
                lax.fori_loop(0, G // 16, scale(u), k * G)
                pltpu.async_copy(stages[u], acc.at[row_v.at[k]], ssems[u],
                                 add=True)
            return 0
        lax.fori_loop(0, NBLK // NSTG, blk, 0)
        for i in range(NSTG):
            k = NBLK - NSTG + i
            pltpu.make_async_copy(
                stages[k % NSTG], acc.at[row_v.at[k]], ssems[k % NSTG]).wait()
        plsc.subcore_barrier()
        pltpu.sync_copy(acc.at[pl.ds(s * RPT, RPT)],
                        dst_ref.at[pl.ds(b * NP + s * RPT, RPT)])

    for (ch, rh, vh, dst_a, dst_b) in (
            (c1_ref, r1_ref, v1_ref, y1_ref, y2_ref),
            (c2_ref, r2_ref, v2_ref, y3_ref, y4_ref)):
        pltpu.sync_copy(ch.at[s], col_v)
        pltpu.sync_copy(rh.at[s], row_v)
        pltpu.sync_copy(vh.at[s], val_v)

        def batch_body(bi, _):
            b = c * BPC + bi
            spmm_pass(x0_ref, dst_a, b)
            spmm_pass(dst_a, dst_b, b)
            return 0
        lax.fori_loop(0, BPC, batch_body, 0)


def _mm_body(x0_ref, y1_ref, y2_ref, y3_ref, y4_ref, w_ref, b_ref, o_ref):
    acc = jnp.dot(x0_ref[0], w_ref[0], preferred_element_type=jnp.float32)
    acc += jnp.dot(y1_ref[0], w_ref[1], preferred_element_type=jnp.float32)
    acc += jnp.dot(y2_ref[0], w_ref[2], preferred_element_type=jnp.float32)
    acc += jnp.dot(y3_ref[0], w_ref[3], preferred_element_type=jnp.float32)
    acc += jnp.dot(y4_ref[0], w_ref[4], preferred_element_type=jnp.float32)
    o_ref[0] = jnp.tanh(acc + b_ref[...])


def _prep_edges(col, row, val):
    cp = jnp.pad(col.reshape(NS, EPT), ((0, 0), (0, EPTP - EPT)))
    rp = jnp.pad(row.reshape(NS, EPT), ((0, 0), (0, EPTP - EPT)))
    vp = jnp.pad(val.reshape(NS, EPT), ((0, 0), (0, EPTP - EPT)))
    return cp, rp.reshape(NS, NBLK, G), vp


def kernel(inputs, state_t, s1_row, s1_col, s1_val, s2_row, s2_col, s2_val,
           weights, biases):
    Bb, Nn, in_dim = inputs.shape
    x_cat = jnp.concatenate([inputs, state_t], axis=2)
    in_size = x_cat.shape[2]
    x0p = jnp.pad(x_cat, ((0, 0), (0, NP - Nn), (0, PADW - in_size)))
    x0f = x0p.reshape(Bb * NP, PADW)
    zeros_hbm = jnp.zeros((RPT, PADW), jnp.float32)

    c1, r1, v1 = _prep_edges(s1_col, s1_row, s1_val)
    c2, r2, v2 = _prep_edges(s2_col, s2_row, s2_val)

    mesh = plsc.VectorSubcoreMesh(core_axis_name="c", subcore_axis_name="s")
    sc = pl.kernel(
        _sc_body,
        out_type=[jax.ShapeDtypeStruct((Bb * NP, PADW), jnp.float32)] * 4,
        mesh=mesh,
        compiler_params=pltpu.CompilerParams(use_tc_tiling_on_sc=False),
        scratch_types=[
            pltpu.VMEM((EPTP,), jnp.int32),
            pltpu.VMEM((NBLK, G), jnp.int32),
            pltpu.VMEM((EPTP,), jnp.float32),
            pltpu.VMEM((G,), jnp.int32),
            pltpu.VMEM((G,), jnp.int32),
            pltpu.VMEM((G,), jnp.int32),
            pltpu.VMEM((G,), jnp.int32),
            pltpu.VMEM((G, PADW), jnp.float32),
            pltpu.VMEM((G, PADW), jnp.float32),
            pltpu.VMEM((G, PADW), jnp.float32),
            pltpu.VMEM((G, PADW), jnp.float32),
            pltpu.VMEM_SHARED((NP, PADW), jnp.float32),
        ] + [pltpu.SemaphoreType.DMA] * 8,
    )
    y1, y2, y3, y4 = sc(x0f, c1, r1, v1, c2, r2, v2, zeros_hbm)

    wm = weights.reshape(in_size, 5, HID)
    wa = jnp.stack([wm[:, 0] - wm[:, 2] - wm[:, 4], wm[:, 1], 2.0 * wm[:, 2],
                    wm[:, 3], 2.0 * wm[:, 4]], axis=0)
    wp = jnp.pad(wa, ((0, 0), (0, PADW - in_size), (0, 0)))

    NB = 1000
    feat_spec = pl.BlockSpec((1, NB, PADW), lambda bb, nn: (bb, nn, 0))
    out = pl.pallas_call(
        _mm_body,
        grid=(Bb, Nn // NB),
        in_specs=[feat_spec] * 5 + [
            pl.BlockSpec((5, PADW, HID), lambda bb, nn: (0, 0, 0)),
            pl.BlockSpec((HID,), lambda bb, nn: (0,)),
        ],
        out_specs=pl.BlockSpec((1, NB, HID), lambda bb, nn: (bb, nn, 0)),
        out_shape=jax.ShapeDtypeStruct((Bb, Nn, HID), jnp.float32),
    )(x0p, y1.reshape(Bb, NP, PADW), y2.reshape(Bb, NP, PADW),
      y3.reshape(Bb, NP, PADW), y4.reshape(Bb, NP, PADW), wp, biases)
    return out
